# baseline (speedup 1.0000x reference)
import functools

import jax
import jax.numpy as jnp
from jax import lax
from jax.experimental import pallas as pl
from jax.experimental.pallas import tpu as pltpu
from jax.experimental.pallas import tpu_sc as plsc

_N = 10000
_E = 320000
_D = 128
_G = 256
_EPS = 1e-5

_NC = 2
_NS = 16
_CHUNK = 125
_EPT = _E // (_NC * _NS)
_NCHUNK = _EPT // _CHUNK
_RPT = 624
_ZR = 16
_IBLK = 20
_NBLK = _NCHUNK // _IBLK

_BN = 2000


def _make_edge_scatter(d):
    mesh = plsc.VectorSubcoreMesh(core_axis_name="c", subcore_axis_name="s",
                                  num_cores=_NC, num_subcores=_NS)

    @functools.partial(
        pl.kernel,
        out_type=jax.ShapeDtypeStruct((_NC, _N, d), jnp.float32),
        mesh=mesh,
        scratch_types=[
            pltpu.VMEM((_IBLK, _CHUNK), jnp.int32),
            pltpu.VMEM((_IBLK, _CHUNK), jnp.int32),
            pltpu.VMEM((_CHUNK, d), jnp.float32),
            pltpu.VMEM((_CHUNK, d), jnp.float32),
            pltpu.VMEM_SHARED((_N, d), jnp.float32),
            pltpu.SemaphoreType.DMA,
            pltpu.SemaphoreType.DMA,
        ],
    )
    def edge_scatter(u_hbm, src_hbm, dst_hbm, out_hbm, idx_s, idx_d, rows_a,
                     rows_b, acc, sem_a, sem_b):
        c = lax.axis_index("c")
        s = lax.axis_index("s")
        w = c * _NS + s

        z16 = jnp.zeros((16,), jnp.float32)
        lanes = d // 16

        def zfill(i, carry):
            rows_a[i // lanes, pl.ds((i % lanes) * 16, 16)] = z16
            return carry

        lax.fori_loop(0, _ZR * lanes, zfill, None)

        last = s == _NS - 1
        base_rows = s * _RPT

        def zcopy(j, carry):
            pltpu.sync_copy(rows_a.at[pl.ds(0, _ZR)],
                            acc.at[pl.ds(base_rows + j * _ZR, _ZR)])
            return carry

        lax.fori_loop(0, jnp.where(last, 40, 39), zcopy, None)
        plsc.subcore_barrier()

        def block(b, carry):
            pltpu.sync_copy(src_hbm.at[w, b], idx_s)
            pltpu.sync_copy(dst_hbm.at[w, b], idx_d)
            pltpu.async_copy(u_hbm.at[idx_s.at[0]], rows_a, sem_a)

            def pair(j, carry2):
                k0 = 2 * j
                pltpu.async_copy(u_hbm.at[idx_s.at[k0 + 1]], rows_b, sem_b)
                pltpu.make_async_copy(u_hbm.at[idx_s.at[k0]], rows_a,
                                      sem_a).wait()
                pltpu.sync_copy(rows_a, acc.at[idx_d.at[k0]], add=True)

                @pl.when(k0 + 2 < _IBLK)
                def _next():
                    pltpu.async_copy(u_hbm.at[idx_s.at[k0 + 2]], rows_a,
                                     sem_a)

                pltpu.make_async_copy(u_hbm.at[idx_s.at[k0 + 1]], rows_b,
                                      sem_b).wait()
                pltpu.sync_copy(rows_b, acc.at[idx_d.at[k0 + 1]], add=True)
                return carry2

            lax.fori_loop(0, _IBLK // 2, pair, None)
            return carry

        lax.fori_loop(0, _NBLK, block, None)
        plsc.subcore_barrier()

        @pl.when(jnp.logical_not(last))
        def _wr():
            pltpu.sync_copy(acc.at[pl.ds(base_rows, _RPT)],
                            out_hbm.at[c, pl.ds(base_rows, _RPT)])

        @pl.when(last)
        def _wr_last():
            pltpu.sync_copy(acc.at[pl.ds(15 * _RPT, _N - 15 * _RPT)],
                            out_hbm.at[c, pl.ds(15 * _RPT, _N - 15 * _RPT)])

    return edge_scatter


_edge128 = _make_edge_scatter(_D)


def _make_deg_scatter():
    d = 16
    mesh = plsc.VectorSubcoreMesh(core_axis_name="c", subcore_axis_name="s",
                                  num_cores=_NC, num_subcores=_NS)

    @functools.partial(
        pl.kernel,
        out_type=jax.ShapeDtypeStruct((_NC, _N, d), jnp.float32),
        mesh=mesh,
        scratch_types=[
            pltpu.VMEM((_IBLK, _CHUNK), jnp.int32),
            pltpu.VMEM((_CHUNK, d), jnp.float32),
            pltpu.VMEM((_ZR, d), jnp.float32),
            pltpu.VMEM_SHARED((_N, d), jnp.float32),
        ],
    )
    def deg_scatter(dst_hbm, out_hbm, idx_d, ones_rows, zbuf, acc):
        c = lax.axis_index("c")
        s = lax.axis_index("s")
        w = c * _NS + s
        one16 = jnp.ones((16,), jnp.float32)
        z16 = jnp.zeros((16,), jnp.float32)

        def ofill(i, carry):
            ones_rows[i, pl.ds(0, 16)] = one16
            return carry

        lax.fori_loop(0, _CHUNK, ofill, None)

        def zfill(i, carry):
            zbuf[i, pl.ds(0, 16)] = z16
            return carry

        lax.fori_loop(0, _ZR, zfill, None)

        last = s == _NS - 1
        base_rows = s * _RPT

        def zcopy(j, carry):
            pltpu.sync_copy(zbuf, acc.at[pl.ds(base_rows + j * _ZR, _ZR)])
            return carry

        lax.fori_loop(0, jnp.where(last, 40, 39), zcopy, None)
        plsc.subcore_barrier()

        def block(b, carry):
            pltpu.sync_copy(dst_hbm.at[w, b], idx_d)

            def echunk(k, carry2):
                pltpu.sync_copy(ones_rows, acc.at[idx_d.at[k]], add=True)
                return carry2

            lax.fori_loop(0, _IBLK, echunk, None)
            return carry

        lax.fori_loop(0, _NBLK, block, None)
        plsc.subcore_barrier()

        @pl.when(jnp.logical_not(last))
        def _wr():
            pltpu.sync_copy(acc.at[pl.ds(base_rows, _RPT)],
                            out_hbm.at[c, pl.ds(base_rows, _RPT)])

        @pl.when(last)
        def _wr_last():
            pltpu.sync_copy(acc.at[pl.ds(15 * _RPT, _N - 15 * _RPT)],
                            out_hbm.at[c, pl.ds(15 * _RPT, _N - 15 * _RPT)])

    return deg_scatter


_deg16 = _make_deg_scatter()


def _tc_first_body(degp_ref, x_ref, w_ref, dinv_ref, u_ref):
    deg = 1.0 + degp_ref[0, :, 0:1] + degp_ref[1, :, 0:1]
    dinv = lax.rsqrt(deg)
    h = jnp.dot(x_ref[...], w_ref[...], preferred_element_type=jnp.float32)
    dinv_ref[...] = dinv
    u_ref[...] = dinv * h


_tc_first = pl.pallas_call(
    _tc_first_body,
    grid=(_N // _BN,),
    in_specs=[
        pl.BlockSpec((_NC, _BN, 16), lambda i: (0, i, 0)),
        pl.BlockSpec((_BN, _D), lambda i: (i, 0)),
        pl.BlockSpec((_D, _D), lambda i: (0, 0)),
    ],
    out_specs=[
        pl.BlockSpec((_BN, 1), lambda i: (i, 0)),
        pl.BlockSpec((_BN, _D), lambda i: (i, 0)),
    ],
    out_shape=[
        jax.ShapeDtypeStruct((_N, 1), jnp.float32),
        jax.ShapeDtypeStruct((_N, _D), jnp.float32),
    ],
)


def _tc_mid_body(p_ref, u_ref, dinv_ref, b_ref, g_ref, be_ref, w_ref,
                 unext_ref):
    dinv = dinv_ref[...]
    out = dinv * (p_ref[0] + p_ref[1] + u_ref[...]) + b_ref[...]
    scale = g_ref[...] * lax.rsqrt(jnp.float32(1.0 + _EPS))
    a = jnp.maximum(out * scale + be_ref[...], 0.0)
    unext_ref[...] = dinv * jnp.dot(a, w_ref[...],
                                    preferred_element_type=jnp.float32)


_tc_mid = pl.pallas_call(
    _tc_mid_body,
    grid=(_N // _BN,),
    in_specs=[
        pl.BlockSpec((_NC, _BN, _D), lambda i: (0, i, 0)),
        pl.BlockSpec((_BN, _D), lambda i: (i, 0)),
        pl.BlockSpec((_BN, 1), lambda i: (i, 0)),
        pl.BlockSpec((1, _D), lambda i: (0, 0)),
        pl.BlockSpec((1, _D), lambda i: (0, 0)),
        pl.BlockSpec((1, _D), lambda i: (0, 0)),
        pl.BlockSpec((_D, _D), lambda i: (0, 0)),
    ],
    out_specs=pl.BlockSpec((_BN, _D), lambda i: (i, 0)),
    out_shape=jax.ShapeDtypeStruct((_N, _D), jnp.float32),
)


def _tc_final_body(p_ref, u_ref, dinv_ref, b_ref, batch_ref, out_ref, acc,
                   cnt):
    i = pl.program_id(0)

    @pl.when(i == 0)
    def _init():
        acc[...] = jnp.zeros_like(acc)
        cnt[...] = jnp.zeros_like(cnt)

    dinv = dinv_ref[...]
    out3 = dinv * (p_ref[0] + p_ref[1] + u_ref[...]) + b_ref[...]
    gids = lax.broadcasted_iota(jnp.int32, (_BN, _G), 1)
    oh = (gids == batch_ref[...]).astype(jnp.float32)
    dn = (((0,), (0,)), ((), ()))
    acc[...] += lax.dot_general(oh, out3, dn,
                                preferred_element_type=jnp.float32)
    cnt[...] += lax.dot_general(oh, jnp.ones((_BN, 1), jnp.float32), dn,
                                preferred_element_type=jnp.float32)

    @pl.when(i == pl.num_programs(0) - 1)
    def _fin():
        out_ref[...] = acc[...] / jnp.maximum(cnt[...], 1.0)


_tc_final = pl.pallas_call(
    _tc_final_body,
    grid=(_N // _BN,),
    in_specs=[
        pl.BlockSpec((_NC, _BN, _D), lambda i: (0, i, 0)),
        pl.BlockSpec((_BN, _D), lambda i: (i, 0)),
        pl.BlockSpec((_BN, 1), lambda i: (i, 0)),
        pl.BlockSpec((1, _D), lambda i: (0, 0)),
        pl.BlockSpec((_BN, 1), lambda i: (i, 0)),
    ],
    out_specs=pl.BlockSpec((_G, _D), lambda i: (0, 0)),
    out_shape=jax.ShapeDtypeStruct((_G, _D), jnp.float32),
    scratch_shapes=[
        pltpu.VMEM((_G, _D), jnp.float32),
        pltpu.VMEM((_G, 1), jnp.float32),
    ],
)


def kernel(x, edge_index, batch, W0, b0, W1, b1, W2, b2, W3, b3, g0, be0, g1,
           be1, g2, be2):
    src = edge_index[0].reshape(_NC * _NS, _NBLK, _IBLK, _CHUNK)
    dst = edge_index[1].reshape(_NC * _NS, _NBLK, _IBLK, _CHUNK)

    degp = _deg16(dst)
    dinv, u = _tc_first(degp, x, W0)

    p = _edge128(u, src, dst)
    u = _tc_mid(p, u, dinv, b0.reshape(1, -1), g0.reshape(1, -1),
                be0.reshape(1, -1), W1)
    p = _edge128(u, src, dst)
    u = _tc_mid(p, u, dinv, b1.reshape(1, -1), g1.reshape(1, -1),
                be1.reshape(1, -1), W2)
    p = _edge128(u, src, dst)
    u = _tc_mid(p, u, dinv, b2.reshape(1, -1), g2.reshape(1, -1),
                be2.reshape(1, -1), W3)
    p = _edge128(u, src, dst)

    return _tc_final(p, u, dinv, b3.reshape(1, -1), batch.reshape(-1, 1))


# speedup vs baseline: 1.0439x; 1.0439x over previous
import functools

import jax
import jax.numpy as jnp
from jax import lax
from jax.experimental import pallas as pl
from jax.experimental.pallas import tpu as pltpu
from jax.experimental.pallas import tpu_sc as plsc

_N = 10000
_E = 320000
_D = 128
_G = 256
_EPS = 1e-5

_NC = 2
_NS = 16
_CHUNK = 80
_EPT = _E // (_NC * _NS)
_NCHUNK = _EPT // _CHUNK
_RPT = 624
_ZR = 16
_IBLK = 25
_NBLK = _NCHUNK // _IBLK

_BN = 2000


def _make_edge_scatter(d):
    mesh = plsc.VectorSubcoreMesh(core_axis_name="c", subcore_axis_name="s",
                                  num_cores=_NC, num_subcores=_NS)

    @functools.partial(
        pl.kernel,
        out_type=jax.ShapeDtypeStruct((_NC, _N, d), jnp.float32),
        mesh=mesh,
        scratch_types=[
            pltpu.VMEM((_IBLK, _CHUNK), jnp.int32),
            pltpu.VMEM((_IBLK, _CHUNK), jnp.int32),
            [pltpu.VMEM((_CHUNK, d), jnp.float32) for _ in range(4)],
            pltpu.VMEM_SHARED((_N, d), jnp.float32),
            [pltpu.SemaphoreType.DMA for _ in range(4)],
        ],
    )
    def edge_scatter(u_hbm, src_hbm, dst_hbm, out_hbm, idx_s, idx_d, rows,
                     acc, gsem):
        c = lax.axis_index("c")
        s = lax.axis_index("s")
        w = c * _NS + s

        z16 = jnp.zeros((16,), jnp.float32)
        lanes = d // 16

        def zfill(i, carry):
            rows[0][i // lanes, pl.ds((i % lanes) * 16, 16)] = z16
            return carry

        lax.fori_loop(0, _ZR * lanes, zfill, None)

        last = s == _NS - 1
        base_rows = s * _RPT

        def zcopy(j, carry):
            pltpu.sync_copy(rows[0].at[pl.ds(0, _ZR)],
                            acc.at[pl.ds(base_rows + j * _ZR, _ZR)])
            return carry

        lax.fori_loop(0, jnp.where(last, 40, 39), zcopy, None)
        plsc.subcore_barrier()

        def block(b, carry):
            pltpu.sync_copy(src_hbm.at[w, b], idx_s)
            pltpu.sync_copy(dst_hbm.at[w, b], idx_d)
            pltpu.async_copy(u_hbm.at[idx_s.at[0]], rows[0], gsem[0])
            pltpu.async_copy(u_hbm.at[idx_s.at[1]], rows[1], gsem[1])

            def quad(j, carry2):
                for i in range(4):
                    k = 4 * j + i
                    kk = k + 2
                    m2 = (i + 2) % 4

                    @pl.when(kk <= _IBLK - 1)
                    def _issue_gather():
                        pltpu.async_copy(u_hbm.at[idx_s.at[kk]], rows[m2],
                                         gsem[m2])

                    pltpu.make_async_copy(u_hbm.at[idx_s.at[k]], rows[i],
                                          gsem[i]).wait()
                    pltpu.sync_copy(rows[i], acc.at[idx_d.at[k]], add=True)
                return carry2

            lax.fori_loop(0, _IBLK // 4, quad, None)
            pltpu.make_async_copy(u_hbm.at[idx_s.at[_IBLK - 1]], rows[0],
                                  gsem[0]).wait()
            pltpu.sync_copy(rows[0], acc.at[idx_d.at[_IBLK - 1]], add=True)
            return carry

        lax.fori_loop(0, _NBLK, block, None)
        plsc.subcore_barrier()

        @pl.when(jnp.logical_not(last))
        def _wr():
            pltpu.sync_copy(acc.at[pl.ds(base_rows, _RPT)],
                            out_hbm.at[c, pl.ds(base_rows, _RPT)])

        @pl.when(last)
        def _wr_last():
            pltpu.sync_copy(acc.at[pl.ds(15 * _RPT, _N - 15 * _RPT)],
                            out_hbm.at[c, pl.ds(15 * _RPT, _N - 15 * _RPT)])

    return edge_scatter


_edge128 = _make_edge_scatter(_D)


def _make_deg_scatter():
    d = 16
    mesh = plsc.VectorSubcoreMesh(core_axis_name="c", subcore_axis_name="s",
                                  num_cores=_NC, num_subcores=_NS)

    @functools.partial(
        pl.kernel,
        out_type=jax.ShapeDtypeStruct((_NC, _N, d), jnp.float32),
        mesh=mesh,
        scratch_types=[
            pltpu.VMEM((_IBLK, _CHUNK), jnp.int32),
            pltpu.VMEM((_CHUNK, d), jnp.float32),
            pltpu.VMEM((_ZR, d), jnp.float32),
            pltpu.VMEM_SHARED((_N, d), jnp.float32),
        ],
    )
    def deg_scatter(dst_hbm, out_hbm, idx_d, ones_rows, zbuf, acc):
        c = lax.axis_index("c")
        s = lax.axis_index("s")
        w = c * _NS + s
        one16 = jnp.ones((16,), jnp.float32)
        z16 = jnp.zeros((16,), jnp.float32)

        def ofill(i, carry):
            ones_rows[i, pl.ds(0, 16)] = one16
            return carry

        lax.fori_loop(0, _CHUNK, ofill, None)

        def zfill(i, carry):
            zbuf[i, pl.ds(0, 16)] = z16
            return carry

        lax.fori_loop(0, _ZR, zfill, None)

        last = s == _NS - 1
        base_rows = s * _RPT

        def zcopy(j, carry):
            pltpu.sync_copy(zbuf, acc.at[pl.ds(base_rows + j * _ZR, _ZR)])
            return carry

        lax.fori_loop(0, jnp.where(last, 40, 39), zcopy, None)
        plsc.subcore_barrier()

        def block(b, carry):
            pltpu.sync_copy(dst_hbm.at[w, b], idx_d)

            def echunk(k, carry2):
                pltpu.sync_copy(ones_rows, acc.at[idx_d.at[k]], add=True)
                return carry2

            lax.fori_loop(0, _IBLK, echunk, None)
            return carry

        lax.fori_loop(0, _NBLK, block, None)
        plsc.subcore_barrier()

        @pl.when(jnp.logical_not(last))
        def _wr():
            pltpu.sync_copy(acc.at[pl.ds(base_rows, _RPT)],
                            out_hbm.at[c, pl.ds(base_rows, _RPT)])

        @pl.when(last)
        def _wr_last():
            pltpu.sync_copy(acc.at[pl.ds(15 * _RPT, _N - 15 * _RPT)],
                            out_hbm.at[c, pl.ds(15 * _RPT, _N - 15 * _RPT)])

    return deg_scatter


_deg16 = _make_deg_scatter()


def _tc_first_body(degp_ref, x_ref, w_ref, dinv_ref, u_ref):
    deg = 1.0 + degp_ref[0, :, 0:1] + degp_ref[1, :, 0:1]
    dinv = lax.rsqrt(deg)
    h = jnp.dot(x_ref[...], w_ref[...], preferred_element_type=jnp.float32)
    dinv_ref[...] = dinv
    u_ref[...] = dinv * h


_tc_first = pl.pallas_call(
    _tc_first_body,
    grid=(_N // _BN,),
    in_specs=[
        pl.BlockSpec((_NC, _BN, 16), lambda i: (0, i, 0)),
        pl.BlockSpec((_BN, _D), lambda i: (i, 0)),
        pl.BlockSpec((_D, _D), lambda i: (0, 0)),
    ],
    out_specs=[
        pl.BlockSpec((_BN, 1), lambda i: (i, 0)),
        pl.BlockSpec((_BN, _D), lambda i: (i, 0)),
    ],
    out_shape=[
        jax.ShapeDtypeStruct((_N, 1), jnp.float32),
        jax.ShapeDtypeStruct((_N, _D), jnp.float32),
    ],
)


def _tc_mid_body(p_ref, u_ref, dinv_ref, b_ref, g_ref, be_ref, w_ref,
                 unext_ref):
    dinv = dinv_ref[...]
    out = dinv * (p_ref[0] + p_ref[1] + u_ref[...]) + b_ref[...]
    scale = g_ref[...] * lax.rsqrt(jnp.float32(1.0 + _EPS))
    a = jnp.maximum(out * scale + be_ref[...], 0.0)
    unext_ref[...] = dinv * jnp.dot(a, w_ref[...],
                                    preferred_element_type=jnp.float32)


_tc_mid = pl.pallas_call(
    _tc_mid_body,
    grid=(_N // _BN,),
    in_specs=[
        pl.BlockSpec((_NC, _BN, _D), lambda i: (0, i, 0)),
        pl.BlockSpec((_BN, _D), lambda i: (i, 0)),
        pl.BlockSpec((_BN, 1), lambda i: (i, 0)),
        pl.BlockSpec((1, _D), lambda i: (0, 0)),
        pl.BlockSpec((1, _D), lambda i: (0, 0)),
        pl.BlockSpec((1, _D), lambda i: (0, 0)),
        pl.BlockSpec((_D, _D), lambda i: (0, 0)),
    ],
    out_specs=pl.BlockSpec((_BN, _D), lambda i: (i, 0)),
    out_shape=jax.ShapeDtypeStruct((_N, _D), jnp.float32),
)


def _tc_final_body(p_ref, u_ref, dinv_ref, b_ref, batch_ref, out_ref, acc,
                   cnt):
    i = pl.program_id(0)

    @pl.when(i == 0)
    def _init():
        acc[...] = jnp.zeros_like(acc)
        cnt[...] = jnp.zeros_like(cnt)

    dinv = dinv_ref[...]
    out3 = dinv * (p_ref[0] + p_ref[1] + u_ref[...]) + b_ref[...]
    gids = lax.broadcasted_iota(jnp.int32, (_BN, _G), 1)
    oh = (gids == batch_ref[...]).astype(jnp.float32)
    dn = (((0,), (0,)), ((), ()))
    acc[...] += lax.dot_general(oh, out3, dn,
                                preferred_element_type=jnp.float32)
    cnt[...] += lax.dot_general(oh, jnp.ones((_BN, 1), jnp.float32), dn,
                                preferred_element_type=jnp.float32)

    @pl.when(i == pl.num_programs(0) - 1)
    def _fin():
        out_ref[...] = acc[...] / jnp.maximum(cnt[...], 1.0)


_tc_final = pl.pallas_call(
    _tc_final_body,
    grid=(_N // _BN,),
    in_specs=[
        pl.BlockSpec((_NC, _BN, _D), lambda i: (0, i, 0)),
        pl.BlockSpec((_BN, _D), lambda i: (i, 0)),
        pl.BlockSpec((_BN, 1), lambda i: (i, 0)),
        pl.BlockSpec((1, _D), lambda i: (0, 0)),
        pl.BlockSpec((_BN, 1), lambda i: (i, 0)),
    ],
    out_specs=pl.BlockSpec((_G, _D), lambda i: (0, 0)),
    out_shape=jax.ShapeDtypeStruct((_G, _D), jnp.float32),
    scratch_shapes=[
        pltpu.VMEM((_G, _D), jnp.float32),
        pltpu.VMEM((_G, 1), jnp.float32),
    ],
)


def kernel(x, edge_index, batch, W0, b0, W1, b1, W2, b2, W3, b3, g0, be0, g1,
           be1, g2, be2):
    src = edge_index[0].reshape(_NC * _NS, _NBLK, _IBLK, _CHUNK)
    dst = edge_index[1].reshape(_NC * _NS, _NBLK, _IBLK, _CHUNK)

    degp = _deg16(dst)
    dinv, u = _tc_first(degp, x, W0)

    p = _edge128(u, src, dst)
    u = _tc_mid(p, u, dinv, b0.reshape(1, -1), g0.reshape(1, -1),
                be0.reshape(1, -1), W1)
    p = _edge128(u, src, dst)
    u = _tc_mid(p, u, dinv, b1.reshape(1, -1), g1.reshape(1, -1),
                be1.reshape(1, -1), W2)
    p = _edge128(u, src, dst)
    u = _tc_mid(p, u, dinv, b2.reshape(1, -1), g2.reshape(1, -1),
                be2.reshape(1, -1), W3)
    p = _edge128(u, src, dst)

    return _tc_final(p, u, dinv, b3.reshape(1, -1), batch.reshape(-1, 1))


# speedup vs baseline: 1.0511x; 1.0069x over previous
import functools

import jax
import jax.numpy as jnp
from jax import lax
from jax.experimental import pallas as pl
from jax.experimental.pallas import tpu as pltpu
from jax.experimental.pallas import tpu_sc as plsc

_N = 10000
_E = 320000
_D = 128
_G = 256
_EPS = 1e-5

_NC = 2
_NS = 16
_CHUNK = 80
_EPT = _E // (_NC * _NS)
_NCHUNK = _EPT // _CHUNK
_RPT = 624
_ZR = 16
_IBLK = 25
_NBLK = _NCHUNK // _IBLK

_BN = 2000


def _make_edge_scatter(d):
    mesh = plsc.VectorSubcoreMesh(core_axis_name="c", subcore_axis_name="s",
                                  num_cores=_NC, num_subcores=_NS)

    @functools.partial(
        pl.kernel,
        out_type=jax.ShapeDtypeStruct((_NC, _N, d), jnp.float32),
        mesh=mesh,
        scratch_types=[
            pltpu.VMEM((_IBLK, _CHUNK), jnp.int32),
            pltpu.VMEM((_IBLK, _CHUNK), jnp.int32),
            [pltpu.VMEM((_CHUNK, d), jnp.float32) for _ in range(4)],
            pltpu.VMEM_SHARED((_N, d), jnp.float32),
            [pltpu.SemaphoreType.DMA for _ in range(4)],
        ],
    )
    def edge_scatter(u_hbm, src_hbm, dst_hbm, out_hbm, idx_s, idx_d, rows,
                     acc, gsem):
        c = lax.axis_index("c")
        s = lax.axis_index("s")
        w = c * _NS + s

        z16 = jnp.zeros((16,), jnp.float32)
        lanes = d // 16

        def zfill(i, carry):
            rows[0][i // lanes, pl.ds((i % lanes) * 16, 16)] = z16
            return carry

        lax.fori_loop(0, _ZR * lanes, zfill, None)

        last = s == _NS - 1
        base_rows = s * _RPT

        def zcopy(j, carry):
            pltpu.sync_copy(rows[0].at[pl.ds(0, _ZR)],
                            acc.at[pl.ds(base_rows + j * _ZR, _ZR)])
            return carry

        lax.fori_loop(0, jnp.where(last, 40, 39), zcopy, None)
        plsc.subcore_barrier()

        def block(b, carry):
            pltpu.sync_copy(src_hbm.at[w, b], idx_s)
            pltpu.sync_copy(dst_hbm.at[w, b], idx_d)
            pltpu.async_copy(u_hbm.at[idx_s.at[0]], rows[0], gsem[0])
            pltpu.async_copy(u_hbm.at[idx_s.at[1]], rows[1], gsem[1])
            pltpu.async_copy(u_hbm.at[idx_s.at[2]], rows[2], gsem[2])

            def quad(j, carry2):
                for i in range(4):
                    k = 4 * j + i
                    kk = k + 3
                    m2 = (i + 3) % 4

                    @pl.when(kk <= _IBLK - 1)
                    def _issue_gather():
                        pltpu.async_copy(u_hbm.at[idx_s.at[kk]], rows[m2],
                                         gsem[m2])

                    pltpu.make_async_copy(u_hbm.at[idx_s.at[k]], rows[i],
                                          gsem[i]).wait()
                    pltpu.sync_copy(rows[i], acc.at[idx_d.at[k]], add=True)
                return carry2

            lax.fori_loop(0, _IBLK // 4, quad, None)
            pltpu.make_async_copy(u_hbm.at[idx_s.at[_IBLK - 1]], rows[0],
                                  gsem[0]).wait()
            pltpu.sync_copy(rows[0], acc.at[idx_d.at[_IBLK - 1]], add=True)
            return carry

        lax.fori_loop(0, _NBLK, block, None)
        plsc.subcore_barrier()

        @pl.when(jnp.logical_not(last))
        def _wr():
            pltpu.sync_copy(acc.at[pl.ds(base_rows, _RPT)],
                            out_hbm.at[c, pl.ds(base_rows, _RPT)])

        @pl.when(last)
        def _wr_last():
            pltpu.sync_copy(acc.at[pl.ds(15 * _RPT, _N - 15 * _RPT)],
                            out_hbm.at[c, pl.ds(15 * _RPT, _N - 15 * _RPT)])

    return edge_scatter


_edge128 = _make_edge_scatter(_D)


def _make_deg_scatter():
    d = 16
    mesh = plsc.VectorSubcoreMesh(core_axis_name="c", subcore_axis_name="s",
                                  num_cores=_NC, num_subcores=_NS)

    @functools.partial(
        pl.kernel,
        out_type=jax.ShapeDtypeStruct((_NC, _N, d), jnp.float32),
        mesh=mesh,
        scratch_types=[
            pltpu.VMEM((_IBLK, _CHUNK), jnp.int32),
            pltpu.VMEM((_CHUNK, d), jnp.float32),
            pltpu.VMEM((_ZR, d), jnp.float32),
            pltpu.VMEM_SHARED((_N, d), jnp.float32),
        ],
    )
    def deg_scatter(dst_hbm, out_hbm, idx_d, ones_rows, zbuf, acc):
        c = lax.axis_index("c")
        s = lax.axis_index("s")
        w = c * _NS + s
        one16 = jnp.ones((16,), jnp.float32)
        z16 = jnp.zeros((16,), jnp.float32)

        def ofill(i, carry):
            ones_rows[i, pl.ds(0, 16)] = one16
            return carry

        lax.fori_loop(0, _CHUNK, ofill, None)

        def zfill(i, carry):
            zbuf[i, pl.ds(0, 16)] = z16
            return carry

        lax.fori_loop(0, _ZR, zfill, None)

        last = s == _NS - 1
        base_rows = s * _RPT

        def zcopy(j, carry):
            pltpu.sync_copy(zbuf, acc.at[pl.ds(base_rows + j * _ZR, _ZR)])
            return carry

        lax.fori_loop(0, jnp.where(last, 40, 39), zcopy, None)
        plsc.subcore_barrier()

        def block(b, carry):
            pltpu.sync_copy(dst_hbm.at[w, b], idx_d)

            def echunk(k, carry2):
                pltpu.sync_copy(ones_rows, acc.at[idx_d.at[k]], add=True)
                return carry2

            lax.fori_loop(0, _IBLK, echunk, None)
            return carry

        lax.fori_loop(0, _NBLK, block, None)
        plsc.subcore_barrier()

        @pl.when(jnp.logical_not(last))
        def _wr():
            pltpu.sync_copy(acc.at[pl.ds(base_rows, _RPT)],
                            out_hbm.at[c, pl.ds(base_rows, _RPT)])

        @pl.when(last)
        def _wr_last():
            pltpu.sync_copy(acc.at[pl.ds(15 * _RPT, _N - 15 * _RPT)],
                            out_hbm.at[c, pl.ds(15 * _RPT, _N - 15 * _RPT)])

    return deg_scatter


_deg16 = _make_deg_scatter()


def _tc_first_body(degp_ref, x_ref, w_ref, dinv_ref, u_ref):
    deg = 1.0 + degp_ref[0, :, 0:1] + degp_ref[1, :, 0:1]
    dinv = lax.rsqrt(deg)
    h = jnp.dot(x_ref[...], w_ref[...], preferred_element_type=jnp.float32)
    dinv_ref[...] = dinv
    u_ref[...] = dinv * h


_tc_first = pl.pallas_call(
    _tc_first_body,
    grid=(_N // _BN,),
    in_specs=[
        pl.BlockSpec((_NC, _BN, 16), lambda i: (0, i, 0)),
        pl.BlockSpec((_BN, _D), lambda i: (i, 0)),
        pl.BlockSpec((_D, _D), lambda i: (0, 0)),
    ],
    out_specs=[
        pl.BlockSpec((_BN, 1), lambda i: (i, 0)),
        pl.BlockSpec((_BN, _D), lambda i: (i, 0)),
    ],
    out_shape=[
        jax.ShapeDtypeStruct((_N, 1), jnp.float32),
        jax.ShapeDtypeStruct((_N, _D), jnp.float32),
    ],
)


def _tc_mid_body(p_ref, u_ref, dinv_ref, b_ref, g_ref, be_ref, w_ref,
                 unext_ref):
    dinv = dinv_ref[...]
    out = dinv * (p_ref[0] + p_ref[1] + u_ref[...]) + b_ref[...]
    scale = g_ref[...] * lax.rsqrt(jnp.float32(1.0 + _EPS))
    a = jnp.maximum(out * scale + be_ref[...], 0.0)
    unext_ref[...] = dinv * jnp.dot(a, w_ref[...],
                                    preferred_element_type=jnp.float32)


_tc_mid = pl.pallas_call(
    _tc_mid_body,
    grid=(_N // _BN,),
    in_specs=[
        pl.BlockSpec((_NC, _BN, _D), lambda i: (0, i, 0)),
        pl.BlockSpec((_BN, _D), lambda i: (i, 0)),
        pl.BlockSpec((_BN, 1), lambda i: (i, 0)),
        pl.BlockSpec((1, _D), lambda i: (0, 0)),
        pl.BlockSpec((1, _D), lambda i: (0, 0)),
        pl.BlockSpec((1, _D), lambda i: (0, 0)),
        pl.BlockSpec((_D, _D), lambda i: (0, 0)),
    ],
    out_specs=pl.BlockSpec((_BN, _D), lambda i: (i, 0)),
    out_shape=jax.ShapeDtypeStruct((_N, _D), jnp.float32),
)


def _tc_final_body(p_ref, u_ref, dinv_ref, b_ref, batch_ref, out_ref, acc,
                   cnt):
    i = pl.program_id(0)

    @pl.when(i == 0)
    def _init():
        acc[...] = jnp.zeros_like(acc)
        cnt[...] = jnp.zeros_like(cnt)

    dinv = dinv_ref[...]
    out3 = dinv * (p_ref[0] + p_ref[1] + u_ref[...]) + b_ref[...]
    gids = lax.broadcasted_iota(jnp.int32, (_BN, _G), 1)
    oh = (gids == batch_ref[...]).astype(jnp.float32)
    dn = (((0,), (0,)), ((), ()))
    acc[...] += lax.dot_general(oh, out3, dn,
                                preferred_element_type=jnp.float32)
    cnt[...] += lax.dot_general(oh, jnp.ones((_BN, 1), jnp.float32), dn,
                                preferred_element_type=jnp.float32)

    @pl.when(i == pl.num_programs(0) - 1)
    def _fin():
        out_ref[...] = acc[...] / jnp.maximum(cnt[...], 1.0)


_tc_final = pl.pallas_call(
    _tc_final_body,
    grid=(_N // _BN,),
    in_specs=[
        pl.BlockSpec((_NC, _BN, _D), lambda i: (0, i, 0)),
        pl.BlockSpec((_BN, _D), lambda i: (i, 0)),
        pl.BlockSpec((_BN, 1), lambda i: (i, 0)),
        pl.BlockSpec((1, _D), lambda i: (0, 0)),
        pl.BlockSpec((_BN, 1), lambda i: (i, 0)),
    ],
    out_specs=pl.BlockSpec((_G, _D), lambda i: (0, 0)),
    out_shape=jax.ShapeDtypeStruct((_G, _D), jnp.float32),
    scratch_shapes=[
        pltpu.VMEM((_G, _D), jnp.float32),
        pltpu.VMEM((_G, 1), jnp.float32),
    ],
)


def kernel(x, edge_index, batch, W0, b0, W1, b1, W2, b2, W3, b3, g0, be0, g1,
           be1, g2, be2):
    src = edge_index[0].reshape(_NC * _NS, _NBLK, _IBLK, _CHUNK)
    dst = edge_index[1].reshape(_NC * _NS, _NBLK, _IBLK, _CHUNK)

    degp = _deg16(dst)
    dinv, u = _tc_first(degp, x, W0)

    p = _edge128(u, src, dst)
    u = _tc_mid(p, u, dinv, b0.reshape(1, -1), g0.reshape(1, -1),
                be0.reshape(1, -1), W1)
    p = _edge128(u, src, dst)
    u = _tc_mid(p, u, dinv, b1.reshape(1, -1), g1.reshape(1, -1),
                be1.reshape(1, -1), W2)
    p = _edge128(u, src, dst)
    u = _tc_mid(p, u, dinv, b2.reshape(1, -1), g2.reshape(1, -1),
                be2.reshape(1, -1), W3)
    p = _edge128(u, src, dst)

    return _tc_final(p, u, dinv, b3.reshape(1, -1), batch.reshape(-1, 1))
